# Initial kernel scaffold; baseline (speedup 1.0000x reference)
import jax
import jax.numpy as jnp
from jax.experimental import pallas as pl


def kernel(x, edge_index, pair_idxs_left, pair_idxs_right, y, W_lin, b_lin, W_gat, a_src, a_dst, b_gat, W1, b1, W2, b2, W3, b3):
    raise NotImplementedError("write your pallas kernel here")


# speedup vs baseline: 3.5578x; 3.5578x over previous
import functools

import jax
import jax.numpy as jnp
from jax import lax
from jax.experimental import pallas as pl
from jax.experimental.pallas import tpu as pltpu
from jax.experimental.pallas import tpu_sc as plsc

_C = 200
_NW = 32


def _mlp_body(x_ref, w1_ref, b1_ref, w2_ref, b2_ref, w3_ref, b3_ref, out_ref):
    h = jnp.dot(x_ref[...], w1_ref[...], preferred_element_type=jnp.float32)
    h = jnp.maximum(h + b1_ref[...], 0.0)
    d = jnp.dot(h, w2_ref[...], preferred_element_type=jnp.float32) + b2_ref[...]
    out_ref[...] = (
        jnp.dot(d, w3_ref[...], preferred_element_type=jnp.float32) + b3_ref[...]
    )


def _mlp(x, W1, b1, W2, b2, W3, b3):
    n, d = x.shape
    h = W1.shape[1]
    h2 = W2.shape[1]
    out_d = W3.shape[1]
    blk = 1000
    return pl.pallas_call(
        _mlp_body,
        grid=(n // blk,),
        in_specs=[
            pl.BlockSpec((blk, d), lambda i: (i, 0)),
            pl.BlockSpec((d, h), lambda i: (0, 0)),
            pl.BlockSpec((1, h), lambda i: (0, 0)),
            pl.BlockSpec((h, h2), lambda i: (0, 0)),
            pl.BlockSpec((1, h2), lambda i: (0, 0)),
            pl.BlockSpec((h2, out_d), lambda i: (0, 0)),
            pl.BlockSpec((1, out_d), lambda i: (0, 0)),
        ],
        out_specs=pl.BlockSpec((blk, out_d), lambda i: (i, 0)),
        out_shape=jax.ShapeDtypeStruct((n, out_d), jnp.float32),
    )(x, W1, b1.reshape(1, -1), W2, b2.reshape(1, -1), W3, b3.reshape(1, -1))


def _sc_gather(x_dnn, x, idx_l, idx_r):
    n_chunks, c = idx_l.shape
    d = x.shape[1]
    mesh = plsc.VectorSubcoreMesh(core_axis_name="c", subcore_axis_name="s")

    @functools.partial(
        pl.kernel,
        mesh=mesh,
        out_type=[
            jax.ShapeDtypeStruct((2, n_chunks, c, d), jnp.float32),
            jax.ShapeDtypeStruct((2, n_chunks, c, d), jnp.float32),
        ],
        scratch_types=[
            pltpu.VMEM((c,), jnp.int32),
            pltpu.VMEM((c, d), jnp.float32),
            pltpu.SemaphoreType.DMA,
        ],
    )
    def k(dnn_hbm, x_hbm, idxl_hbm, idxr_hbm, emb_hbm, feat_hbm, idx_v, rows_v, sem):
        wid = lax.axis_index("s") * 2 + lax.axis_index("c")
        n_mine = (n_chunks - wid + _NW - 1) // _NW

        for table, idx_hbm, out_hbm, side in (
            (dnn_hbm, idxl_hbm, emb_hbm, 0),
            (dnn_hbm, idxr_hbm, emb_hbm, 1),
            (x_hbm, idxl_hbm, feat_hbm, 0),
            (x_hbm, idxr_hbm, feat_hbm, 1),
        ):

            def body(i, carry, idx_hbm=idx_hbm, table=table, out_hbm=out_hbm,
                     side=side):
                ch = wid + i * _NW
                pltpu.sync_copy(idx_hbm.at[ch], idx_v)
                pltpu.async_copy(table.at[idx_v], rows_v, sem).wait()
                pltpu.sync_copy(rows_v, out_hbm.at[side, ch])
                return carry

            lax.fori_loop(0, n_mine, body, 0)

    return k(x_dnn, x, idx_l, idx_r)


def kernel(x, edge_index, pair_idxs_left, pair_idxs_right, y, W_lin, b_lin,
           W_gat, a_src, a_dst, b_gat, W1, b1, W2, b2, W3, b3):
    p = pair_idxs_left.shape[0]
    x_dnn = _mlp(x, W1, b1, W2, b2, W3, b3)
    idx_l = pair_idxs_left.reshape(-1, _C)
    idx_r = pair_idxs_right.reshape(-1, _C)
    emb, feat = _sc_gather(x_dnn, x, idx_l, idx_r)
    return (emb.reshape(2, p, -1), feat.reshape(2, p, -1), y)
